# baseline (speedup 1.0000x reference)
import jax
import jax.numpy as jnp
import numpy as np
from jax import lax
from jax.experimental import pallas as pl
from jax.experimental.pallas import tpu as pltpu
from jax.sharding import Mesh, PartitionSpec as P

try:
    _shard_map = jax.shard_map
except AttributeError:
    from jax.experimental.shard_map import shard_map as _shard_map

G0, G1 = 64, 128
KN = G0 * G1
TN = 256


def _som_kernel(x_ref, w_ref, out_ref, wh_ref, wl_ref, w2_ref):
    @pl.when(pl.program_id(0) == 0)
    def _():
        wf = w_ref[...]
        w2_ref[...] = jnp.sum(wf * wf, axis=0, keepdims=True)
        wh = wf.astype(jnp.bfloat16)
        wh_ref[...] = wh
        wl_ref[...] = (wf - wh.astype(jnp.float32)).astype(jnp.bfloat16)

    x = x_ref[...]
    x2 = jnp.sum(x * x, axis=1, keepdims=True)
    xs = -2.0 * x
    xh = xs.astype(jnp.bfloat16)
    xl = (xs - xh.astype(jnp.float32)).astype(jnp.bfloat16)
    dn = (((1,), (0,)), ((), ()))
    wh, wl = wh_ref[...], wl_ref[...]
    dot = (lax.dot_general(xh, wh, dn, preferred_element_type=jnp.float32)
           + lax.dot_general(xh, wl, dn, preferred_element_type=jnp.float32)
           + lax.dot_general(xl, wh, dn, preferred_element_type=jnp.float32))
    a = dot + w2_ref[...]
    m = jnp.min(a, axis=1, keepdims=True)
    kiota = lax.broadcasted_iota(jnp.int32, (TN, KN), 1)
    sel = jnp.where(a == m, kiota, KN)
    idx = jnp.min(sel, axis=1, keepdims=True)
    p0 = idx // G1
    p1 = idx - p0 * G1
    iu = lax.broadcasted_iota(jnp.int32, (TN, G0), 1)
    iv = lax.broadcasted_iota(jnp.int32, (TN, G1), 1)
    du = (iu - p0).astype(jnp.float32)
    dv = (iv - p1).astype(jnp.float32)
    u = jnp.exp(-(du * du))
    v = jnp.exp(-(dv * dv))
    wgt = jnp.concatenate([v * u[:, i:i + 1] for i in range(G0)], axis=1)
    sq = jnp.maximum(a + x2, 0.0)
    out_ref[...] = jnp.sum(wgt * sq, axis=1, keepdims=True)


def _som_shard(x, w):
    n, d = x.shape
    out = pl.pallas_call(
        _som_kernel,
        grid=(n // TN,),
        in_specs=[
            pl.BlockSpec((TN, d), lambda i: (i, 0)),
            pl.BlockSpec((d, KN), lambda i: (0, 0)),
        ],
        out_specs=pl.BlockSpec((TN, 1), lambda i: (i, 0)),
        out_shape=jax.ShapeDtypeStruct((n, 1), jnp.float32),
        scratch_shapes=[
            pltpu.VMEM((d, KN), jnp.bfloat16),
            pltpu.VMEM((d, KN), jnp.bfloat16),
            pltpu.VMEM((1, KN), jnp.float32),
        ],
    )(x, w)
    return out[:, 0]


def kernel(x, w):
    n = x.shape[0]
    devs = jax.devices()
    nd = max(d for d in range(1, len(devs) + 1) if n % (d * TN) == 0)
    if nd > 1:
        mesh = Mesh(np.array(devs[:nd]), ("dp",))
        f = _shard_map(_som_shard, mesh=mesh,
                       in_specs=(P("dp", None), P(None, None)),
                       out_specs=P("dp"), check_vma=False)
        return f(x, w)
    return _som_shard(x, w)


# speedup vs baseline: 3.2286x; 3.2286x over previous
import jax
import jax.numpy as jnp
from jax import lax
from jax.experimental import pallas as pl

G0, G1 = 64, 128
KN = G0 * G1
TN = 256


def _wprep_kernel(w_ref, wh_ref, wl_ref, w2_ref):
    wf = w_ref[...]
    wh = wf.astype(jnp.bfloat16)
    wh_ref[...] = wh
    wl_ref[...] = (wf - wh.astype(jnp.float32)).astype(jnp.bfloat16)
    w2_ref[...] = jnp.sum(wf * wf, axis=0, keepdims=True)


def _som_kernel(x_ref, wh_ref, wl_ref, w2_ref, out_ref):
    x = x_ref[...]
    x2 = jnp.sum(x * x, axis=1, keepdims=True)
    xs = -2.0 * x
    xh = xs.astype(jnp.bfloat16)
    xl = (xs - xh.astype(jnp.float32)).astype(jnp.bfloat16)
    dn = (((1,), (0,)), ((), ()))
    wh, wl = wh_ref[...], wl_ref[...]
    dot = (lax.dot_general(xh, wh, dn, preferred_element_type=jnp.float32)
           + lax.dot_general(xh, wl, dn, preferred_element_type=jnp.float32)
           + lax.dot_general(xl, wh, dn, preferred_element_type=jnp.float32))
    a = dot + w2_ref[...]
    m = jnp.min(a, axis=1, keepdims=True)
    kiota = lax.broadcasted_iota(jnp.int32, (TN, KN), 1)
    sel = jnp.where(a == m, kiota, KN)
    idx = jnp.min(sel, axis=1, keepdims=True)
    p0 = idx // G1
    p1 = idx - p0 * G1
    iu = lax.broadcasted_iota(jnp.int32, (TN, G0), 1)
    iv = lax.broadcasted_iota(jnp.int32, (TN, G1), 1)
    du = (iu - p0).astype(jnp.float32)
    dv = (iv - p1).astype(jnp.float32)
    u = jnp.exp(-(du * du))
    v = jnp.exp(-(dv * dv))
    wgt = jnp.concatenate([v * u[:, i:i + 1] for i in range(G0)], axis=1)
    sq = jnp.maximum(a + x2, 0.0)
    out_ref[...] = jnp.sum(wgt * sq, axis=1, keepdims=True)


def kernel(x, w):
    n, d = x.shape
    wh, wl, w2 = pl.pallas_call(
        _wprep_kernel,
        out_shape=(
            jax.ShapeDtypeStruct((d, KN), jnp.bfloat16),
            jax.ShapeDtypeStruct((d, KN), jnp.bfloat16),
            jax.ShapeDtypeStruct((1, KN), jnp.float32),
        ),
    )(w)
    out = pl.pallas_call(
        _som_kernel,
        grid=(n // TN,),
        in_specs=[
            pl.BlockSpec((TN, d), lambda i: (i, 0)),
            pl.BlockSpec((d, KN), lambda i: (0, 0)),
            pl.BlockSpec((d, KN), lambda i: (0, 0)),
            pl.BlockSpec((1, KN), lambda i: (0, 0)),
        ],
        out_specs=pl.BlockSpec((TN, 1), lambda i: (i, 0)),
        out_shape=jax.ShapeDtypeStruct((n, 1), jnp.float32),
    )(x, wh, wl, w2)
    return out[:, 0]


# speedup vs baseline: 3.3582x; 1.0401x over previous
import jax
import jax.numpy as jnp
from jax import lax
from jax.experimental import pallas as pl
from jax.experimental.pallas import tpu as pltpu

G0, G1 = 64, 128
KN = G0 * G1
TN = 256


def _som_kernel(x_ref, w_ref, out_ref, wh_ref, wl_ref, w2_ref):
    @pl.when(pl.program_id(0) == 0)
    def _():
        wf = w_ref[...]
        w2_ref[...] = jnp.sum(wf * wf, axis=0, keepdims=True)
        wh = wf.astype(jnp.bfloat16)
        wh_ref[...] = wh
        wl_ref[...] = (wf - wh.astype(jnp.float32)).astype(jnp.bfloat16)

    x = x_ref[...]
    x2 = jnp.sum(x * x, axis=1, keepdims=True)
    xs = -2.0 * x
    xh = xs.astype(jnp.bfloat16)
    xl = (xs - xh.astype(jnp.float32)).astype(jnp.bfloat16)
    dn = (((1,), (0,)), ((), ()))
    wh, wl = wh_ref[...], wl_ref[...]
    dot = (lax.dot_general(xh, wh, dn, preferred_element_type=jnp.float32)
           + lax.dot_general(xh, wl, dn, preferred_element_type=jnp.float32)
           + lax.dot_general(xl, wh, dn, preferred_element_type=jnp.float32))
    a = dot + w2_ref[...]
    m = jnp.min(a, axis=1, keepdims=True)
    kiota = lax.broadcasted_iota(jnp.int32, (TN, KN), 1)
    sel = jnp.where(a == m, kiota, KN)
    idx = jnp.min(sel, axis=1, keepdims=True)
    p0 = idx // G1
    p1 = idx - p0 * G1
    iu = lax.broadcasted_iota(jnp.int32, (TN, G0), 1)
    iv = lax.broadcasted_iota(jnp.int32, (TN, G1), 1)
    du = (iu - p0).astype(jnp.float32)
    dv = (iv - p1).astype(jnp.float32)
    u = jnp.exp(-(du * du))
    v = jnp.exp(-(dv * dv))
    wgt = jnp.concatenate([v * u[:, i:i + 1] for i in range(G0)], axis=1)
    sq = jnp.maximum(a + x2, 0.0)
    out_ref[...] = jnp.sum(wgt * sq, axis=1, keepdims=True)


def kernel(x, w):
    n, d = x.shape
    out = pl.pallas_call(
        _som_kernel,
        grid=(n // TN,),
        in_specs=[
            pl.BlockSpec((TN, d), lambda i: (i, 0)),
            pl.BlockSpec((d, KN), lambda i: (0, 0)),
        ],
        out_specs=pl.BlockSpec((TN, 1), lambda i: (i, 0)),
        out_shape=jax.ShapeDtypeStruct((n, 1), jnp.float32),
        scratch_shapes=[
            pltpu.VMEM((d, KN), jnp.bfloat16),
            pltpu.VMEM((d, KN), jnp.bfloat16),
            pltpu.VMEM((1, KN), jnp.float32),
        ],
    )(x, w)
    return out[:, 0]
